# baseline (speedup 1.0000x reference)
import jax
import jax.numpy as jnp
from jax.experimental import pallas as pl

N = 4096
IN_F = 512
OUT_F = 256
H = 4
HID = OUT_F // H

BM = 512
BI = 512
BJ = 512


def _proj_kernel(vert_ref, w_ref, acomb_ref, g_ref, ss_ref):
    g = jnp.dot(vert_ref[...], w_ref[...], preferred_element_type=jnp.float32)
    g_ref[...] = g
    ss_ref[...] = jnp.dot(g, acomb_ref[...], preferred_element_type=jnp.float32)


def _attn_kernel(ssrc_ref, sdst_row_ref, sdst_col_ref, tmax_ref, g_ref, out_ref):
    c = tmax_ref[0, 0, 0]
    s_col = ssrc_ref[0]
    x = s_col + c
    a_scl = jnp.exp(0.8 * jnp.minimum(x, 0.0))
    b_scl = jnp.exp(-0.8 * jnp.maximum(x, 0.0))

    acc = jnp.zeros((BI, 2 * HID), jnp.float32)
    accp = jnp.zeros((BI, 1), jnp.float32)
    accq = jnp.zeros((BI, 1), jnp.float32)
    qg_tot = jnp.zeros((1, HID), jnp.float32)
    q_tot = jnp.zeros((1, 1), jnp.float32)

    for jc in range(N // BJ):
        t_row = sdst_row_ref[0][:, jc * BJ:(jc + 1) * BJ]
        t_col = sdst_col_ref[0][jc * BJ:(jc + 1) * BJ, :]
        gj = g_ref[0][jc * BJ:(jc + 1) * BJ, :]
        p_row = jnp.exp(t_row - c)
        q_row = jnp.exp(0.2 * (t_row - c))
        p_col = jnp.exp(t_col - c)
        q_col = jnp.exp(0.2 * (t_col - c))
        cmat = jnp.concatenate([p_col * gj, q_col * gj], axis=1)
        mask_b = ((s_col + t_row) > 0.0).astype(jnp.bfloat16)
        acc = acc + jnp.dot(mask_b, cmat.astype(jnp.bfloat16),
                            preferred_element_type=jnp.float32)
        mask = mask_b.astype(jnp.float32)
        accp = accp + jnp.sum(mask * p_row, axis=1, keepdims=True)
        accq = accq + jnp.sum(mask * q_row, axis=1, keepdims=True)
        qg_tot = qg_tot + jnp.sum(q_col * gj, axis=0, keepdims=True)
        q_tot = q_tot + jnp.sum(q_row, axis=1, keepdims=True)

    numer = a_scl * acc[:, :HID] + b_scl * (qg_tot - acc[:, HID:])
    denom = a_scl * accp + b_scl * (q_tot - accq)
    o = numer / denom
    out_ref[0] = jnp.where(o > 0.0, o, jnp.exp(jnp.minimum(o, 0.0)) - 1.0)


def kernel(vert, W, a_src, a_dst):
    idx = jnp.arange(OUT_F)
    head = idx // HID
    sel = (head[:, None] == jnp.arange(H)[None, :]).astype(jnp.float32)
    acomb = jnp.concatenate(
        [sel * a_src.reshape(-1)[:, None], sel * a_dst.reshape(-1)[:, None]],
        axis=1)

    g, ss = pl.pallas_call(
        _proj_kernel,
        grid=(N // BM,),
        in_specs=[
            pl.BlockSpec((BM, IN_F), lambda i: (i, 0)),
            pl.BlockSpec((IN_F, OUT_F), lambda i: (0, 0)),
            pl.BlockSpec((OUT_F, 2 * H), lambda i: (0, 0)),
        ],
        out_specs=[
            pl.BlockSpec((BM, OUT_F), lambda i: (i, 0)),
            pl.BlockSpec((BM, 2 * H), lambda i: (i, 0)),
        ],
        out_shape=[
            jax.ShapeDtypeStruct((N, OUT_F), jnp.float32),
            jax.ShapeDtypeStruct((N, 2 * H), jnp.float32),
        ],
    )(vert, W, acomb)

    ssrc = ss[:, :H].T
    sdst = ss[:, H:].T
    ssrc_col = ssrc.reshape(H, N, 1)
    sdst_row = sdst.reshape(H, 1, N)
    sdst_col = sdst.reshape(H, N, 1)
    tmax = jnp.max(sdst, axis=1).reshape(H, 1, 1)
    g_h = g.reshape(N, H, HID).transpose(1, 0, 2)

    out = pl.pallas_call(
        _attn_kernel,
        grid=(H, N // BI),
        in_specs=[
            pl.BlockSpec((1, BI, 1), lambda h, ti: (h, ti, 0)),
            pl.BlockSpec((1, 1, N), lambda h, ti: (h, 0, 0)),
            pl.BlockSpec((1, N, 1), lambda h, ti: (h, 0, 0)),
            pl.BlockSpec((1, 1, 1), lambda h, ti: (h, 0, 0)),
            pl.BlockSpec((1, N, HID), lambda h, ti: (h, 0, 0)),
        ],
        out_specs=pl.BlockSpec((1, BI, HID), lambda h, ti: (h, ti, 0)),
        out_shape=jax.ShapeDtypeStruct((H, N, HID), jnp.float32),
    )(ssrc_col, sdst_row, sdst_col, tmax, g_h)
    return out.transpose(1, 0, 2).reshape(N, OUT_F)


# speedup vs baseline: 1.1829x; 1.1829x over previous
import jax
import jax.numpy as jnp
from jax.experimental import pallas as pl

N = 4096
IN_F = 512
OUT_F = 256
H = 4
HID = OUT_F // H

BM = 512
BI = 512
BJ = 512


def _proj_kernel(vert_ref, w_ref, acomb_ref, g_ref, ss_ref):
    g = jnp.dot(vert_ref[...], w_ref[...], preferred_element_type=jnp.float32)
    g_ref[...] = g
    ss_ref[...] = jnp.dot(g, acomb_ref[...], preferred_element_type=jnp.float32)


def _attn_kernel(ssrc_ref, sdst_row_ref, sdst_col_ref, tmax_ref, g_ref, out_ref):
    c = tmax_ref[0, 0, 0]
    s_col = ssrc_ref[0]
    x = s_col + c
    a_scl = jnp.exp(0.8 * jnp.minimum(x, 0.0))
    b_scl = jnp.exp(-0.8 * jnp.maximum(x, 0.0))

    acc = jnp.zeros((BI, 2 * HID), jnp.float32)
    accpq = jnp.zeros((BI, 2), jnp.float32)
    qg_tot = jnp.zeros((1, HID), jnp.float32)
    q_tot = jnp.zeros((1, 1), jnp.float32)

    for jc in range(N // BJ):
        t_row = sdst_row_ref[0][:, jc * BJ:(jc + 1) * BJ]
        t_col = sdst_col_ref[0][jc * BJ:(jc + 1) * BJ, :]
        gj = g_ref[0][jc * BJ:(jc + 1) * BJ, :]
        p_col = jnp.exp(t_col - c)
        q_col = jnp.exp(0.2 * (t_col - c))
        qg = q_col * gj
        cmat = jnp.concatenate([p_col * gj, qg], axis=1)
        pq = jnp.concatenate([p_col, q_col], axis=1)
        mask_b = ((s_col + t_row) > 0.0).astype(jnp.bfloat16)
        acc = acc + jnp.dot(mask_b, cmat.astype(jnp.bfloat16),
                            preferred_element_type=jnp.float32)
        accpq = accpq + jnp.dot(mask_b, pq.astype(jnp.bfloat16),
                                preferred_element_type=jnp.float32)
        qg_tot = qg_tot + jnp.sum(qg, axis=0, keepdims=True)
        q_tot = q_tot + jnp.sum(q_col, axis=0, keepdims=True)

    numer = a_scl * acc[:, :HID] + b_scl * (qg_tot - acc[:, HID:])
    denom = a_scl * accpq[:, 0:1] + b_scl * (q_tot - accpq[:, 1:2])
    o = numer / denom
    out_ref[0] = jnp.where(o > 0.0, o, jnp.exp(jnp.minimum(o, 0.0)) - 1.0)


def kernel(vert, W, a_src, a_dst):
    idx = jnp.arange(OUT_F)
    head = idx // HID
    sel = (head[:, None] == jnp.arange(H)[None, :]).astype(jnp.float32)
    acomb = jnp.concatenate(
        [sel * a_src.reshape(-1)[:, None], sel * a_dst.reshape(-1)[:, None]],
        axis=1)

    g, ss = pl.pallas_call(
        _proj_kernel,
        grid=(N // BM,),
        in_specs=[
            pl.BlockSpec((BM, IN_F), lambda i: (i, 0)),
            pl.BlockSpec((IN_F, OUT_F), lambda i: (0, 0)),
            pl.BlockSpec((OUT_F, 2 * H), lambda i: (0, 0)),
        ],
        out_specs=[
            pl.BlockSpec((BM, OUT_F), lambda i: (i, 0)),
            pl.BlockSpec((BM, 2 * H), lambda i: (i, 0)),
        ],
        out_shape=[
            jax.ShapeDtypeStruct((N, OUT_F), jnp.float32),
            jax.ShapeDtypeStruct((N, 2 * H), jnp.float32),
        ],
    )(vert, W, acomb)

    ssrc = ss[:, :H].T
    sdst = ss[:, H:].T
    ssrc_col = ssrc.reshape(H, N, 1)
    sdst_row = sdst.reshape(H, 1, N)
    sdst_col = sdst.reshape(H, N, 1)
    tmax = jnp.max(sdst, axis=1).reshape(H, 1, 1)
    g_h = g.reshape(N, H, HID).transpose(1, 0, 2)

    out = pl.pallas_call(
        _attn_kernel,
        grid=(H, N // BI),
        in_specs=[
            pl.BlockSpec((1, BI, 1), lambda h, ti: (h, ti, 0)),
            pl.BlockSpec((1, 1, N), lambda h, ti: (h, 0, 0)),
            pl.BlockSpec((1, N, 1), lambda h, ti: (h, 0, 0)),
            pl.BlockSpec((1, 1, 1), lambda h, ti: (h, 0, 0)),
            pl.BlockSpec((1, N, HID), lambda h, ti: (h, 0, 0)),
        ],
        out_specs=pl.BlockSpec((1, BI, HID), lambda h, ti: (h, ti, 0)),
        out_shape=jax.ShapeDtypeStruct((H, N, HID), jnp.float32),
    )(ssrc_col, sdst_row, sdst_col, tmax, g_h)
    return out.transpose(1, 0, 2).reshape(N, OUT_F)


# speedup vs baseline: 1.2606x; 1.0657x over previous
import jax
import jax.numpy as jnp
from jax.experimental import pallas as pl

N = 4096
IN_F = 512
OUT_F = 256
H = 4
HID = OUT_F // H

BM = 512
BI = 512
BJ = 512


def _proj_kernel(vert_ref, w_ref, acomb_ref, g_ref, ss_ref):
    g = jnp.dot(vert_ref[...], w_ref[...], preferred_element_type=jnp.float32)
    g_ref[...] = g
    ss_ref[...] = jnp.dot(g, acomb_ref[...], preferred_element_type=jnp.float32)


def _prep_kernel(g_ref, sdst_col_ref, tmax_ref, cmat_ref, pq_ref,
                 qgtot_ref, qtot_ref):
    c = tmax_ref[0, 0, 0]
    t = sdst_col_ref[0]
    g = g_ref[0]
    p = jnp.exp(t - c)
    q = jnp.exp(0.2 * (t - c))
    qg = q * g
    cmat_ref[0] = jnp.concatenate([p * g, qg], axis=1).astype(jnp.bfloat16)
    pq_ref[0] = jnp.concatenate([p, q], axis=1).astype(jnp.bfloat16)
    qgtot_ref[0] = jnp.sum(qg, axis=0, keepdims=True)
    qtot_ref[0] = jnp.sum(q, axis=0, keepdims=True)


def _attn_kernel(ssrc_ref, sdst_row_ref, tmax_ref, cmat_ref, pq_ref,
                 qgtot_ref, qtot_ref, out_ref):
    c = tmax_ref[0, 0, 0]
    s_col = ssrc_ref[0]
    x = s_col + c
    a_scl = jnp.exp(0.8 * jnp.minimum(x, 0.0))
    b_scl = jnp.exp(-0.8 * jnp.maximum(x, 0.0))
    s_col_b = s_col.astype(jnp.bfloat16)
    t_row_b = sdst_row_ref[0].astype(jnp.bfloat16)

    acc = jnp.zeros((BI, 2 * HID), jnp.float32)
    accpq = jnp.zeros((BI, 2), jnp.float32)
    for jc in range(N // BJ):
        tb = t_row_b[:, jc * BJ:(jc + 1) * BJ]
        mask_b = ((s_col_b + tb) > 0).astype(jnp.bfloat16)
        acc = acc + jnp.dot(mask_b, cmat_ref[0][jc * BJ:(jc + 1) * BJ, :],
                            preferred_element_type=jnp.float32)
        accpq = accpq + jnp.dot(mask_b, pq_ref[0][jc * BJ:(jc + 1) * BJ, :],
                                preferred_element_type=jnp.float32)

    qg_tot = qgtot_ref[0]
    q_tot = qtot_ref[0]
    numer = a_scl * acc[:, :HID] + b_scl * (qg_tot - acc[:, HID:])
    denom = a_scl * accpq[:, 0:1] + b_scl * (q_tot - accpq[:, 1:2])
    o = numer / denom
    out_ref[0] = jnp.where(o > 0.0, o, jnp.exp(jnp.minimum(o, 0.0)) - 1.0)


def kernel(vert, W, a_src, a_dst):
    idx = jnp.arange(OUT_F)
    head = idx // HID
    sel = (head[:, None] == jnp.arange(H)[None, :]).astype(jnp.float32)
    acomb = jnp.concatenate(
        [sel * a_src.reshape(-1)[:, None], sel * a_dst.reshape(-1)[:, None]],
        axis=1)

    g, ss = pl.pallas_call(
        _proj_kernel,
        grid=(N // BM,),
        in_specs=[
            pl.BlockSpec((BM, IN_F), lambda i: (i, 0)),
            pl.BlockSpec((IN_F, OUT_F), lambda i: (0, 0)),
            pl.BlockSpec((OUT_F, 2 * H), lambda i: (0, 0)),
        ],
        out_specs=[
            pl.BlockSpec((BM, OUT_F), lambda i: (i, 0)),
            pl.BlockSpec((BM, 2 * H), lambda i: (i, 0)),
        ],
        out_shape=[
            jax.ShapeDtypeStruct((N, OUT_F), jnp.float32),
            jax.ShapeDtypeStruct((N, 2 * H), jnp.float32),
        ],
    )(vert, W, acomb)

    ssrc = ss[:, :H].T
    sdst = ss[:, H:].T
    ssrc_col = ssrc.reshape(H, N, 1)
    sdst_row = sdst.reshape(H, 1, N)
    sdst_col = sdst.reshape(H, N, 1)
    tmax = jnp.max(sdst, axis=1).reshape(H, 1, 1)
    g_h = g.reshape(N, H, HID).transpose(1, 0, 2)

    cmat, pq, qgtot, qtot = pl.pallas_call(
        _prep_kernel,
        grid=(H,),
        in_specs=[
            pl.BlockSpec((1, N, HID), lambda h: (h, 0, 0)),
            pl.BlockSpec((1, N, 1), lambda h: (h, 0, 0)),
            pl.BlockSpec((1, 1, 1), lambda h: (h, 0, 0)),
        ],
        out_specs=[
            pl.BlockSpec((1, N, 2 * HID), lambda h: (h, 0, 0)),
            pl.BlockSpec((1, N, 2), lambda h: (h, 0, 0)),
            pl.BlockSpec((1, 1, HID), lambda h: (h, 0, 0)),
            pl.BlockSpec((1, 1, 1), lambda h: (h, 0, 0)),
        ],
        out_shape=[
            jax.ShapeDtypeStruct((H, N, 2 * HID), jnp.bfloat16),
            jax.ShapeDtypeStruct((H, N, 2), jnp.bfloat16),
            jax.ShapeDtypeStruct((H, 1, HID), jnp.float32),
            jax.ShapeDtypeStruct((H, 1, 1), jnp.float32),
        ],
    )(g_h, sdst_col, tmax)

    out = pl.pallas_call(
        _attn_kernel,
        grid=(H, N // BI),
        in_specs=[
            pl.BlockSpec((1, BI, 1), lambda h, ti: (h, ti, 0)),
            pl.BlockSpec((1, 1, N), lambda h, ti: (h, 0, 0)),
            pl.BlockSpec((1, 1, 1), lambda h, ti: (h, 0, 0)),
            pl.BlockSpec((1, N, 2 * HID), lambda h, ti: (h, 0, 0)),
            pl.BlockSpec((1, N, 2), lambda h, ti: (h, 0, 0)),
            pl.BlockSpec((1, 1, HID), lambda h, ti: (h, 0, 0)),
            pl.BlockSpec((1, 1, 1), lambda h, ti: (h, 0, 0)),
        ],
        out_specs=pl.BlockSpec((1, BI, HID), lambda h, ti: (h, ti, 0)),
        out_shape=jax.ShapeDtypeStruct((H, N, HID), jnp.float32),
    )(ssrc_col, sdst_row, tmax, cmat, pq, qgtot, qtot)
    return out.transpose(1, 0, 2).reshape(N, OUT_F)


# speedup vs baseline: 1.7922x; 1.4217x over previous
import jax
import jax.numpy as jnp
from jax import lax
from jax.experimental import pallas as pl

N = 4096
IN_F = 512
OUT_F = 256
H = 4
HID = OUT_F // H
H2 = H // 2

BM = 512
BI = 512
BJ = 512

def _proj_kernel(vert_ref, w_ref, acomb_ref, g_ref, ss_ref):
    g = jnp.dot(vert_ref[...], w_ref[...], preferred_element_type=jnp.float32)
    g_ref[...] = g
    ss_ref[...] = jnp.dot(g, acomb_ref[...], preferred_element_type=jnp.float32)


def _step_mask_bf16(x_b):
    return jnp.where(x_b > 0, jnp.bfloat16(1.0), jnp.bfloat16(0.0))


def _prep_kernel(g_ref, sdst_col_ref, tmax_ref, cmat_ref, pq_ref,
                 qgtot_ref, qtot_ref):
    for hp in range(2):
        c = tmax_ref[0, 0, hp]
        t = sdst_col_ref[0][:, hp:hp + 1]
        gh = g_ref[:, hp * HID:(hp + 1) * HID]
        p = jnp.exp(t - c)
        q = jnp.exp(0.2 * (t - c))
        qg = q * gh
        cmat_ref[:, hp * 2 * HID:hp * 2 * HID + HID] = (p * gh).astype(jnp.bfloat16)
        cmat_ref[:, hp * 2 * HID + HID:(hp + 1) * 2 * HID] = qg.astype(jnp.bfloat16)
        pq_ref[0, :, 2 * hp:2 * hp + 1] = p.astype(jnp.bfloat16)
        pq_ref[0, :, 2 * hp + 1:2 * hp + 2] = q.astype(jnp.bfloat16)
        qgtot_ref[0, hp:hp + 1, :] = jnp.sum(qg, axis=0, keepdims=True)
        qtot_ref[0, :, hp:hp + 1] = jnp.sum(q, axis=0, keepdims=True)


def _attn_kernel(ssrc_ref, sdst_row_ref, tmax_ref, cmat_ref, pq_ref,
                 qgtot_ref, qtot_ref, out_ref):
    for hp in range(2):
        c = tmax_ref[0, 0, hp]
        s_col = ssrc_ref[0][:, hp:hp + 1]
        x = s_col + c
        a_scl = jnp.exp(0.8 * jnp.minimum(x, 0.0))
        b_scl = jnp.exp(-0.8 * jnp.maximum(x, 0.0))
        s_col_b = s_col.astype(jnp.bfloat16)
        t_row_b = sdst_row_ref[0][hp:hp + 1, :].astype(jnp.bfloat16)

        acc = jnp.zeros((BI, 2 * HID), jnp.float32)
        accpq = jnp.zeros((BI, 2), jnp.float32)
        for jc in range(N // BJ):
            tb = t_row_b[:, jc * BJ:(jc + 1) * BJ]
            mask_b = _step_mask_bf16(s_col_b + tb)
            acc = acc + jnp.dot(
                mask_b,
                cmat_ref[jc * BJ:(jc + 1) * BJ, hp * 2 * HID:(hp + 1) * 2 * HID],
                preferred_element_type=jnp.float32)
            accpq = accpq + jnp.dot(
                mask_b, pq_ref[0][jc * BJ:(jc + 1) * BJ, 2 * hp:2 * hp + 2],
                preferred_element_type=jnp.float32)

        qg_tot = qgtot_ref[0][hp:hp + 1, :]
        q_tot = qtot_ref[0][:, hp:hp + 1]
        numer = a_scl * acc[:, :HID] + b_scl * (qg_tot - acc[:, HID:])
        denom = a_scl * accpq[:, 0:1] + b_scl * (q_tot - accpq[:, 1:2])
        o = numer / denom
        out_ref[:, hp * HID:(hp + 1) * HID] = jnp.where(
            o > 0.0, o, jnp.exp(jnp.minimum(o, 0.0)) - 1.0)


def kernel(vert, W, a_src, a_dst):
    idx = jnp.arange(OUT_F)
    head = idx // HID
    sel = (head[:, None] == jnp.arange(H)[None, :]).astype(jnp.float32)
    acomb = jnp.concatenate(
        [sel * a_src.reshape(-1)[:, None], sel * a_dst.reshape(-1)[:, None]],
        axis=1)

    g, ss = pl.pallas_call(
        _proj_kernel,
        grid=(N // BM,),
        in_specs=[
            pl.BlockSpec((BM, IN_F), lambda i: (i, 0)),
            pl.BlockSpec((IN_F, OUT_F), lambda i: (0, 0)),
            pl.BlockSpec((OUT_F, 2 * H), lambda i: (0, 0)),
        ],
        out_specs=[
            pl.BlockSpec((BM, OUT_F), lambda i: (i, 0)),
            pl.BlockSpec((BM, 2 * H), lambda i: (i, 0)),
        ],
        out_shape=[
            jax.ShapeDtypeStruct((N, OUT_F), jnp.float32),
            jax.ShapeDtypeStruct((N, 2 * H), jnp.float32),
        ],
    )(vert, W, acomb)

    ssrc2 = ss[:, :H].reshape(N, H2, 2).transpose(1, 0, 2)
    sdst2_col = ss[:, H:].reshape(N, H2, 2).transpose(1, 0, 2)
    sdst2_row = sdst2_col.transpose(0, 2, 1)
    tmax2 = jnp.max(sdst2_col, axis=1, keepdims=True)

    cmat, pq, qgtot, qtot = pl.pallas_call(
        _prep_kernel,
        grid=(H2,),
        in_specs=[
            pl.BlockSpec((N, 2 * HID), lambda h: (0, h)),
            pl.BlockSpec((1, N, 2), lambda h: (h, 0, 0)),
            pl.BlockSpec((1, 1, 2), lambda h: (h, 0, 0)),
        ],
        out_specs=[
            pl.BlockSpec((N, 4 * HID), lambda h: (0, h)),
            pl.BlockSpec((1, N, 4), lambda h: (h, 0, 0)),
            pl.BlockSpec((1, 2, HID), lambda h: (h, 0, 0)),
            pl.BlockSpec((1, 1, 2), lambda h: (h, 0, 0)),
        ],
        out_shape=[
            jax.ShapeDtypeStruct((N, 2 * OUT_F), jnp.bfloat16),
            jax.ShapeDtypeStruct((H2, N, 4), jnp.bfloat16),
            jax.ShapeDtypeStruct((H2, 2, HID), jnp.float32),
            jax.ShapeDtypeStruct((H2, 1, 2), jnp.float32),
        ],
    )(g, sdst2_col, tmax2)

    out = pl.pallas_call(
        _attn_kernel,
        grid=(H2, N // BI),
        in_specs=[
            pl.BlockSpec((1, BI, 2), lambda h, ti: (h, ti, 0)),
            pl.BlockSpec((1, 2, N), lambda h, ti: (h, 0, 0)),
            pl.BlockSpec((1, 1, 2), lambda h, ti: (h, 0, 0)),
            pl.BlockSpec((N, 4 * HID), lambda h, ti: (0, h)),
            pl.BlockSpec((1, N, 4), lambda h, ti: (h, 0, 0)),
            pl.BlockSpec((1, 2, HID), lambda h, ti: (h, 0, 0)),
            pl.BlockSpec((1, 1, 2), lambda h, ti: (h, 0, 0)),
        ],
        out_specs=pl.BlockSpec((BI, 2 * HID), lambda h, ti: (ti, h)),
        out_shape=jax.ShapeDtypeStruct((N, OUT_F), jnp.float32),
    )(ssrc2, sdst2_row, tmax2, cmat, pq, qgtot, qtot)
    return out


# speedup vs baseline: 2.1711x; 1.2114x over previous
import jax
import jax.numpy as jnp
from jax import lax
from jax.experimental import pallas as pl
from jax.experimental.pallas import tpu as pltpu

N = 4096
IN_F = 512
OUT_F = 256
H = 4
HID = OUT_F // H

BM = 512
BI = 1024
NPROJ = N // BM
NATT = N // BI


def _fused_kernel(vert_ref, w_ref, acomb_ref, out_ref,
                  g_s, ss_s, ssr_s, tmax_s, cmat_s, pq_s, qgtot_s, qtot_s):
    k = pl.program_id(0)

    @pl.when(k < NPROJ)
    def _proj():
        g = jnp.dot(vert_ref[...], w_ref[...],
                    preferred_element_type=jnp.float32)
        ss_row = lax.dot_general(
            acomb_ref[...], g, (((0,), (1,)), ((), ())),
            preferred_element_type=jnp.float32)
        ss = jnp.dot(g, acomb_ref[...],
                     preferred_element_type=jnp.float32)
        g_s[pl.ds(k * BM, BM), :] = g
        ss_s[pl.ds(k * BM, BM), :] = ss
        ssr_s[k] = ss_row
        m = jnp.max(ss, axis=0, keepdims=True)

        @pl.when(k == 0)
        def _():
            tmax_s[...] = m

        @pl.when(k > 0)
        def _():
            tmax_s[...] = jnp.maximum(tmax_s[...], m)

    @pl.when(k == NPROJ)
    def _prep():
        for h in range(H):
            c = tmax_s[0, H + h]
            t = ss_s[:, H + h:H + h + 1]
            gh = g_s[:, h * HID:(h + 1) * HID]
            p = jnp.exp(t - c)
            q = jnp.exp(0.2 * (t - c))
            qg = q * gh
            cmat_s[:, h * 2 * HID:h * 2 * HID + HID] = (
                p * gh).astype(jnp.bfloat16)
            cmat_s[:, h * 2 * HID + HID:(h + 1) * 2 * HID] = (
                qg.astype(jnp.bfloat16))
            pq_s[:, 2 * h:2 * h + 1] = p.astype(jnp.bfloat16)
            pq_s[:, 2 * h + 1:2 * h + 2] = q.astype(jnp.bfloat16)
            qgtot_s[h:h + 1, :] = jnp.sum(qg, axis=0, keepdims=True)
            qtot_s[:, h:h + 1] = jnp.sum(q, axis=0, keepdims=True)

    @pl.when(k >= NPROJ)
    def _attn():
        i0 = (k - NPROJ) * BI
        for h in range(H):
            c = tmax_s[0, H + h]
            s_col = ss_s[pl.ds(i0, BI), h:h + 1]
            x = s_col + c
            a_scl = jnp.exp(0.8 * jnp.minimum(x, 0.0))
            b_scl = jnp.exp(-0.8 * jnp.maximum(x, 0.0))
            ns_col_b = (-s_col).astype(jnp.bfloat16)

            acc = jnp.zeros((BI, 2 * HID), jnp.float32)
            accpq = jnp.zeros((BI, 2), jnp.float32)
            for jc in range(NPROJ):
                tb = ssr_s[jc, H + h:H + h + 1, :].astype(jnp.bfloat16)
                mask_b = jnp.where(tb > ns_col_b,
                                   jnp.bfloat16(1.0), jnp.bfloat16(0.0))
                acc = acc + jnp.dot(
                    mask_b,
                    cmat_s[jc * BM:(jc + 1) * BM,
                           h * 2 * HID:(h + 1) * 2 * HID],
                    preferred_element_type=jnp.float32)
                accpq = accpq + jnp.dot(
                    mask_b, pq_s[jc * BM:(jc + 1) * BM, 2 * h:2 * h + 2],
                    preferred_element_type=jnp.float32)

            numer = a_scl * acc[:, :HID] + b_scl * (qgtot_s[h:h + 1, :]
                                                    - acc[:, HID:])
            denom = a_scl * accpq[:, 0:1] + b_scl * (qtot_s[:, h:h + 1]
                                                     - accpq[:, 1:2])
            o = numer / denom
            out_ref[:, h * HID:(h + 1) * HID] = jnp.where(
                o > 0.0, o, jnp.exp(jnp.minimum(o, 0.0)) - 1.0)


def kernel(vert, W, a_src, a_dst):
    idx = jnp.arange(OUT_F)
    head = idx // HID
    sel = (head[:, None] == jnp.arange(H)[None, :]).astype(jnp.float32)
    acomb = jnp.concatenate(
        [sel * a_src.reshape(-1)[:, None], sel * a_dst.reshape(-1)[:, None]],
        axis=1)

    out = pl.pallas_call(
        _fused_kernel,
        grid=(NPROJ + NATT,),
        in_specs=[
            pl.BlockSpec((BM, IN_F), lambda k: (jnp.minimum(k, NPROJ - 1), 0)),
            pl.BlockSpec((IN_F, OUT_F), lambda k: (0, 0)),
            pl.BlockSpec((OUT_F, 2 * H), lambda k: (0, 0)),
        ],
        out_specs=pl.BlockSpec(
            (BI, OUT_F), lambda k: (jnp.maximum(k - NPROJ, 0), 0)),
        out_shape=jax.ShapeDtypeStruct((N, OUT_F), jnp.float32),
        scratch_shapes=[
            pltpu.VMEM((N, OUT_F), jnp.float32),
            pltpu.VMEM((N, 2 * H), jnp.float32),
            pltpu.VMEM((NPROJ, 2 * H, BM), jnp.float32),
            pltpu.VMEM((1, 2 * H), jnp.float32),
            pltpu.VMEM((N, 2 * OUT_F), jnp.bfloat16),
            pltpu.VMEM((N, 2 * H), jnp.bfloat16),
            pltpu.VMEM((H, HID), jnp.float32),
            pltpu.VMEM((1, H), jnp.float32),
        ],
    )(vert, W, acomb)
    return out


# speedup vs baseline: 2.3383x; 1.0770x over previous
import jax
import jax.numpy as jnp
from jax import lax
from jax.experimental import pallas as pl
from jax.experimental.pallas import tpu as pltpu

N = 4096
IN_F = 512
OUT_F = 256
H = 4
HID = OUT_F // H

BM = 512
BI = 1024
NPROJ = N // BM
NATT = N // BI


def _fused_kernel(vert_ref, w_ref, acomb_ref, out_ref,
                  g_s, ss_s, ssr_s, tmax_s, cmat_s, pq_s, qgtot_s, qtot_s):
    k = pl.program_id(0)

    @pl.when(k < NPROJ)
    def _proj():
        g = jnp.dot(vert_ref[...], w_ref[...],
                    preferred_element_type=jnp.float32)
        ss_row = lax.dot_general(
            acomb_ref[...], g, (((0,), (1,)), ((), ())),
            preferred_element_type=jnp.float32)
        ss = jnp.dot(g, acomb_ref[...],
                     preferred_element_type=jnp.float32)
        g_s[pl.ds(k * BM, BM), :] = g
        ss_s[pl.ds(k * BM, BM), :] = ss
        ssr_s[k] = ss_row
        m = jnp.max(ss, axis=0, keepdims=True)

        @pl.when(k == 0)
        def _():
            tmax_s[...] = m

        @pl.when(k > 0)
        def _():
            tmax_s[...] = jnp.maximum(tmax_s[...], m)

    @pl.when(k == NPROJ)
    def _prep():
        c_row = tmax_s[0:1, H:2 * H]
        t_all = ss_s[:, H:2 * H]
        p_all = jnp.exp(t_all - c_row)
        q_all = jnp.exp(0.2 * (t_all - c_row))
        pq_s[:, 0:H] = p_all.astype(jnp.bfloat16)
        pq_s[:, H:2 * H] = q_all.astype(jnp.bfloat16)
        qtot_s[...] = jnp.sum(q_all, axis=0, keepdims=True)
        for h in range(H):
            ghb = g_s[:, h * HID:(h + 1) * HID].astype(jnp.bfloat16)
            qgb = q_all[:, h:h + 1].astype(jnp.bfloat16) * ghb
            cmat_s[:, h * 2 * HID:h * 2 * HID + HID] = (
                p_all[:, h:h + 1].astype(jnp.bfloat16) * ghb)
            cmat_s[:, h * 2 * HID + HID:(h + 1) * 2 * HID] = qgb
            qgtot_s[h:h + 1, :] = jnp.sum(qgb, axis=0, keepdims=True,
                                          dtype=jnp.float32)

    @pl.when(k >= NPROJ)
    def _attn():
        i0 = (k - NPROJ) * BI
        for h in range(H):
            c = tmax_s[0, H + h]
            s_col = ss_s[pl.ds(i0, BI), h:h + 1]
            x = s_col + c
            a_scl = jnp.exp(0.8 * jnp.minimum(x, 0.0))
            b_scl = jnp.exp(-0.8 * jnp.maximum(x, 0.0))
            ns_col_b = (-s_col).astype(jnp.bfloat16)

            acc = jnp.zeros((BI, 2 * HID), jnp.float32)
            accpq = jnp.zeros((BI, 2 * H), jnp.float32)
            for jc in range(NPROJ):
                tb = ssr_s[jc, H + h:H + h + 1, :].astype(jnp.bfloat16)
                mask_b = jnp.where(tb > ns_col_b,
                                   jnp.bfloat16(1.0), jnp.bfloat16(0.0))
                acc = acc + jnp.dot(
                    mask_b,
                    cmat_s[jc * BM:(jc + 1) * BM,
                           h * 2 * HID:(h + 1) * 2 * HID],
                    preferred_element_type=jnp.float32)
                accpq = accpq + jnp.dot(
                    mask_b, pq_s[jc * BM:(jc + 1) * BM, :],
                    preferred_element_type=jnp.float32)

            numer = a_scl * acc[:, :HID] + b_scl * (qgtot_s[h:h + 1, :]
                                                    - acc[:, HID:])
            denom = a_scl * accpq[:, h:h + 1] + b_scl * (
                qtot_s[:, h:h + 1] - accpq[:, H + h:H + h + 1])
            o = numer / denom
            out_ref[:, h * HID:(h + 1) * HID] = jnp.where(
                o > 0.0, o, jnp.exp(jnp.minimum(o, 0.0)) - 1.0)


def kernel(vert, W, a_src, a_dst):
    idx = jnp.arange(OUT_F)
    head = idx // HID
    sel = (head[:, None] == jnp.arange(H)[None, :]).astype(jnp.float32)
    acomb = jnp.concatenate(
        [sel * a_src.reshape(-1)[:, None], sel * a_dst.reshape(-1)[:, None]],
        axis=1)

    out = pl.pallas_call(
        _fused_kernel,
        grid=(NPROJ + NATT,),
        in_specs=[
            pl.BlockSpec((BM, IN_F), lambda k: (jnp.minimum(k, NPROJ - 1), 0)),
            pl.BlockSpec((IN_F, OUT_F), lambda k: (0, 0)),
            pl.BlockSpec((OUT_F, 2 * H), lambda k: (0, 0)),
        ],
        out_specs=pl.BlockSpec(
            (BI, OUT_F), lambda k: (jnp.maximum(k - NPROJ, 0), 0)),
        out_shape=jax.ShapeDtypeStruct((N, OUT_F), jnp.float32),
        scratch_shapes=[
            pltpu.VMEM((N, OUT_F), jnp.float32),
            pltpu.VMEM((N, 2 * H), jnp.float32),
            pltpu.VMEM((NPROJ, 2 * H, BM), jnp.float32),
            pltpu.VMEM((1, 2 * H), jnp.float32),
            pltpu.VMEM((N, 2 * OUT_F), jnp.bfloat16),
            pltpu.VMEM((N, 2 * H), jnp.bfloat16),
            pltpu.VMEM((H, HID), jnp.float32),
            pltpu.VMEM((1, H), jnp.float32),
        ],
    )(vert, W, acomb)
    return out
